# baseline (speedup 1.0000x reference)
import functools

import jax
import jax.numpy as jnp
from jax.experimental import pallas as pl
from jax.experimental.pallas import tpu as pltpu

N = 65536
E = 8
D_EMB = 32
D_SHS = 48
D_IN = 92
D_H = 256
D_OUT = 59

_BLK = 2048


def _dense_body(ts_ref, m_ref, s_ref, r_ref, o_ref, shs_ref, emb_ref, t_ref,
                tstart_ref, tend_ref, seg_ref, W1_ref, b1_ref, W2_ref, b2_ref,
                m_out, s_out, r_out, o_out, shs_out, mask_out):
    ts = ts_ref[0, 0]
    m = m_ref[...]
    s = s_ref[...]
    r = r_ref[...]
    o = o_ref[...]
    shs = shs_ref[...]
    x = jnp.concatenate(
        [m, s, r, o, shs, emb_ref[...], t_ref[...]], axis=1)
    seg = seg_ref[...]
    delta = jnp.zeros((x.shape[0], D_OUT), jnp.float32)
    for e in range(E):
        h = jnp.tanh(
            jnp.dot(x, W1_ref[e], preferred_element_type=jnp.float32)
            + b1_ref[e:e + 1, :])
        d = (jnp.dot(h, W2_ref[e], preferred_element_type=jnp.float32)
             + b2_ref[e:e + 1, :])
        delta = delta + jnp.where(seg == e, d, 0.0)
    active = (ts >= tstart_ref[...]) & (ts < tend_ref[...])
    m_out[...] = jnp.where(active, m + delta[:, 0:3], m)
    s_out[...] = jnp.where(active, s + delta[:, 3:6], s)
    r_out[...] = jnp.where(active, r + delta[:, 6:10], r)
    o_out[...] = jnp.where(active, o + delta[:, 10:11], -100.0)
    shs_out[...] = jnp.where(active, shs + delta[:, 11:59], shs)
    mask_out[...] = active.astype(jnp.float32)


@jax.jit
def _run_dense(ts, means3D, scales, rotations, opacity, shs2, embeddings,
               time, tstart, tend, seg, W1, b1, W2, b2):
    nblk = N // _BLK
    row = lambda i: (i, 0)
    fixed2 = lambda i: (0, 0)
    fixed3 = lambda i: (0, 0, 0)
    outs = pl.pallas_call(
        _dense_body,
        grid=(nblk,),
        in_specs=[
            pl.BlockSpec((1, 1), fixed2),
            pl.BlockSpec((_BLK, 3), row),
            pl.BlockSpec((_BLK, 3), row),
            pl.BlockSpec((_BLK, 4), row),
            pl.BlockSpec((_BLK, 1), row),
            pl.BlockSpec((_BLK, D_SHS), row),
            pl.BlockSpec((_BLK, D_EMB), row),
            pl.BlockSpec((_BLK, 1), row),
            pl.BlockSpec((_BLK, 1), row),
            pl.BlockSpec((_BLK, 1), row),
            pl.BlockSpec((_BLK, 1), row),
            pl.BlockSpec((E, D_IN, D_H), fixed3),
            pl.BlockSpec((E, D_H), fixed2),
            pl.BlockSpec((E, D_H, D_OUT), fixed3),
            pl.BlockSpec((E, D_OUT), fixed2),
        ],
        out_specs=[
            pl.BlockSpec((_BLK, 3), row),
            pl.BlockSpec((_BLK, 3), row),
            pl.BlockSpec((_BLK, 4), row),
            pl.BlockSpec((_BLK, 1), row),
            pl.BlockSpec((_BLK, D_SHS), row),
            pl.BlockSpec((_BLK, 1), row),
        ],
        out_shape=[
            jax.ShapeDtypeStruct((N, 3), jnp.float32),
            jax.ShapeDtypeStruct((N, 3), jnp.float32),
            jax.ShapeDtypeStruct((N, 4), jnp.float32),
            jax.ShapeDtypeStruct((N, 1), jnp.float32),
            jax.ShapeDtypeStruct((N, D_SHS), jnp.float32),
            jax.ShapeDtypeStruct((N, 1), jnp.float32),
        ],
        compiler_params=pltpu.CompilerParams(
            dimension_semantics=("parallel",)),
    )(ts, means3D, scales, rotations, opacity, shs2, embeddings, time,
      tstart, tend, seg, W1, b1, W2, b2)
    return outs


def kernel(means3D, scales, rotations, opacity, shs, time, embeddings,
           seg_id_g, t_start_g, t_end_g, W1, b1, W2, b2):
    n = means3D.shape[0]
    shs2 = shs.reshape(n, D_SHS)
    seg = seg_id_g.astype(jnp.int32).reshape(n, 1)
    tstart = t_start_g.reshape(n, 1)
    tend = t_end_g.reshape(n, 1)
    ts = time.reshape(-1)[0].reshape(1, 1)
    m_f, s_f, r_f, o_f, shs_f, mask_f = _run_dense(
        ts, means3D, scales, rotations, opacity, shs2, embeddings, time,
        tstart, tend, seg, W1, b1, W2, b2)
    active_mask = mask_f.reshape(n).astype(bool)
    return (m_f, s_f, r_f, o_f, shs_f.reshape(n, 16, 3), active_mask)
